# baseline (speedup 1.0000x reference)
import functools

import jax
import jax.numpy as jnp
from jax import lax
from jax.experimental import pallas as pl
from jax.experimental.pallas import tpu as pltpu
from jax.experimental.pallas import tpu_sc as plsc

B, D, K, TOPK = 256, 768, 100000, 5
KB = 5000
NBLK = K // KB
EPS = 1e-8
NEG_INF = float("-inf")

NFULL = KB // 128
TAILW = KB - NFULL * 128
NCH = NFULL + (1 if TAILW else 0)
RUNCH = NCH


def _topk_body(content_ref, keys_ref, spk_ref, tgt_ref, idx_out, w_out,
               run_vals, run_idx, qn_s):
    pid = pl.program_id(0)

    @pl.when(pid == 0)
    def _init():
        run_vals[...] = jnp.full((128, B), NEG_INF, jnp.float32)
        run_idx[...] = jnp.zeros((128, B), jnp.int32)
        content = content_ref[...]
        qn_s[...] = content / (jnp.sqrt(jnp.sum(content * content, axis=1,
                                                keepdims=True)) + EPS)

    qn = qn_s[...]
    keys = keys_ref[...]
    ssq = jnp.sum(keys * keys, axis=1, keepdims=True)
    kn = keys / (jnp.sqrt(ssq) + EPS)

    sims = lax.dot_general(kn, qn, (((1,), (1,)), ((), ())),
                           preferred_element_type=jnp.float32)

    spk = spk_ref[...]
    tgt = tgt_ref[...]
    masked = jnp.where(spk == tgt, sims, NEG_INF)

    run_old = run_vals[...]
    ri_old = run_idx[...]
    sub = lax.broadcasted_iota(jnp.int32, (128, B), 0)

    def chunk(j):
        if j == RUNCH:
            return run_old
        c = masked[j * 128:min((j + 1) * 128, KB), :]
        if c.shape[0] < 128:
            c = jnp.concatenate(
                [c, jnp.full((128 - c.shape[0], B), NEG_INF, jnp.float32)],
                axis=0)
        return c

    M = run_old
    G = jnp.full((128, B), RUNCH, jnp.int32)
    for j in range(NCH):
        c = chunk(j)
        upd = c > M
        M = jnp.where(upd, c, M)
        G = jnp.where(upd, j, G)

    nv = jnp.full((128, B), NEG_INF, jnp.float32)
    ni = jnp.zeros((128, B), jnp.int32)
    for t in range(TOPK):
        v = jnp.max(M, axis=0, keepdims=True)
        eq = M == v
        l = jnp.min(jnp.where(eq, sub, 128), axis=0, keepdims=True)
        onehot = sub == l
        g = jnp.sum(jnp.where(onehot, G, 0), axis=0, keepdims=True)
        ri_sel = jnp.sum(jnp.where(onehot, ri_old, 0), axis=0, keepdims=True)
        gidx = jnp.where(g < RUNCH, pid * KB + g * 128 + l, ri_sel)
        nv = jnp.where(sub == t, v, nv)
        ni = jnp.where(sub == t, gidx, ni)
        best = jnp.full((1, B), NEG_INF, jnp.float32)
        bestj = jnp.full((1, B), RUNCH, jnp.int32)
        for j in range(NCH + 1):
            colv = jnp.max(jnp.where(onehot, chunk(j), NEG_INF),
                           axis=0, keepdims=True)
            colv = jnp.where(colv < v, colv, NEG_INF)
            upd = colv > best
            best = jnp.where(upd, colv, best)
            bestj = jnp.where(upd, j, bestj)
        M = jnp.where(onehot, best, M)
        G = jnp.where(onehot, bestj, G)

    run_vals[...] = nv
    run_idx[...] = ni

    @pl.when(pid == NBLK - 1)
    def _fini():
        finite = (nv > NEG_INF) & (sub < TOPK)
        valid = jnp.where(finite, 1.0, 0.0)
        denom = jnp.maximum(jnp.sum(valid, axis=0, keepdims=True), 1.0)
        w = valid / denom
        idx_out[...] = ni[:8, :]
        w_out[...] = w[:8, :]


def _masked_topk(content, keys, spk, tgt):
    spk2 = spk.reshape(K, 1)
    tgt2 = tgt.reshape(1, B)
    return pl.pallas_call(
        _topk_body,
        grid=(NBLK,),
        in_specs=[
            pl.BlockSpec((B, D), lambda i: (0, 0)),
            pl.BlockSpec((KB, D), lambda i: (i, 0)),
            pl.BlockSpec((KB, 1), lambda i: (i, 0)),
            pl.BlockSpec((1, B), lambda i: (0, 0)),
        ],
        out_specs=[
            pl.BlockSpec((8, B), lambda i: (0, 0)),
            pl.BlockSpec((8, B), lambda i: (0, 0)),
        ],
        out_shape=[
            jax.ShapeDtypeStruct((8, B), jnp.int32),
            jax.ShapeDtypeStruct((8, B), jnp.float32),
        ],
        scratch_shapes=[
            pltpu.VMEM((128, B), jnp.float32),
            pltpu.VMEM((128, B), jnp.int32),
            pltpu.VMEM((B, D), jnp.float32),
        ],
        compiler_params=pltpu.CompilerParams(
            dimension_semantics=("arbitrary",)),
    )(content, keys, spk2, tgt2)


def _sc_gather(table, idx_flat):
    info = plsc.get_sparse_core_info()
    nc, ns = info.num_cores, info.num_subcores
    nw = nc * ns
    n = idx_flat.shape[0]
    b_per_w = n // nw
    mesh = plsc.VectorSubcoreMesh(core_axis_name="c", subcore_axis_name="s")

    @functools.partial(
        pl.kernel,
        mesh=mesh,
        out_type=jax.ShapeDtypeStruct((n, D), jnp.float32),
        scratch_types=[
            pltpu.VMEM((b_per_w,), jnp.int32),
            pltpu.VMEM((b_per_w, D), jnp.float32),
            pltpu.SemaphoreType.DMA,
        ],
    )
    def gather_kernel(table_hbm, idx_hbm, out_hbm, idx_v, rows_v, sem):
        wid = lax.axis_index("s") * nc + lax.axis_index("c")
        base = wid * b_per_w
        pltpu.sync_copy(idx_hbm.at[pl.ds(base, b_per_w)], idx_v)
        pltpu.async_copy(table_hbm.at[idx_v], rows_v, sem).wait()
        pltpu.sync_copy(rows_v, out_hbm.at[pl.ds(base, b_per_w)])

    return gather_kernel(table, idx_flat)


def _mlp_body(content_ref, g_ref, w_ref, W1_ref, b1_ref, W2_ref, b2_ref,
              out_ref):
    content = content_ref[...]
    w8 = w_ref[...]
    rmean = jnp.zeros((B, D), jnp.float32)
    sumw = jnp.zeros((B, 1), jnp.float32)
    for t in range(TOPK):
        wt = w8[t].reshape(B, 1)
        rmean = rmean + g_ref[t] * wt
        sumw = sumw + wt
    has_any = sumw > 0.5

    W1 = W1_ref[...]
    h = lax.dot_general(content, W1[:, :D], (((1,), (1,)), ((), ())),
                        preferred_element_type=jnp.float32)
    h = h + lax.dot_general(rmean, W1[:, D:], (((1,), (1,)), ((), ())),
                            preferred_element_type=jnp.float32)
    h = jnp.maximum(h + b1_ref[...], 0.0)
    out = lax.dot_general(h, W2_ref[...], (((1,), (1,)), ((), ())),
                          preferred_element_type=jnp.float32)
    out = out + b2_ref[...]
    out_ref[...] = jnp.where(has_any, out, content)


def _mlp(content, gathered, w8, W1, b1, W2, b2):
    return pl.pallas_call(
        _mlp_body,
        out_shape=jax.ShapeDtypeStruct((B, D), jnp.float32),
    )(content, gathered, w8, W1, b1.reshape(1, D), W2, b2.reshape(1, D))


@jax.jit
def kernel(content_features, training_features, W1, b1, W2, b2,
           target_speaker_id, speaker_ids):
    top_idx8, top_w8 = _masked_topk(content_features, training_features,
                                    speaker_ids.astype(jnp.int32),
                                    target_speaker_id.astype(jnp.int32))
    idx_flat = top_idx8[:TOPK].reshape(B * TOPK)
    gathered = _sc_gather(training_features, idx_flat)
    g3 = gathered.reshape(TOPK, B, D)
    return _mlp(content_features, g3, top_w8, W1, b1, W2, b2)


# speedup vs baseline: 1.0049x; 1.0049x over previous
import functools

import jax
import jax.numpy as jnp
from jax import lax
from jax.experimental import pallas as pl
from jax.experimental.pallas import tpu as pltpu
from jax.experimental.pallas import tpu_sc as plsc

B, D, K, TOPK = 256, 768, 100000, 5
KB = 4000
NBLK = K // KB
EPS = 1e-8
NEG_INF = float("-inf")

NFULL = KB // 128
TAILW = KB - NFULL * 128
NCH = NFULL + (1 if TAILW else 0)
RUNCH = NCH


def _topk_body(content_ref, keys_ref, spk_ref, tgt_ref, idx_out, w_out,
               run_vals, run_idx, qn_s):
    pid = pl.program_id(0)

    @pl.when(pid == 0)
    def _init():
        run_vals[...] = jnp.full((128, B), NEG_INF, jnp.float32)
        run_idx[...] = jnp.zeros((128, B), jnp.int32)
        content = content_ref[...]
        qn_s[...] = content / (jnp.sqrt(jnp.sum(content * content, axis=1,
                                                keepdims=True)) + EPS)

    qn = qn_s[...]
    keys = keys_ref[...]
    ssq = jnp.sum(keys * keys, axis=1, keepdims=True)
    kn = keys / (jnp.sqrt(ssq) + EPS)

    sims = lax.dot_general(kn, qn, (((1,), (1,)), ((), ())),
                           preferred_element_type=jnp.float32)

    spk = spk_ref[...]
    tgt = tgt_ref[...]
    masked = jnp.where(spk == tgt, sims, NEG_INF)

    run_old = run_vals[...]
    ri_old = run_idx[...]
    sub = lax.broadcasted_iota(jnp.int32, (128, B), 0)

    def chunk(j):
        if j == RUNCH:
            return run_old
        c = masked[j * 128:min((j + 1) * 128, KB), :]
        if c.shape[0] < 128:
            c = jnp.concatenate(
                [c, jnp.full((128 - c.shape[0], B), NEG_INF, jnp.float32)],
                axis=0)
        return c

    M = run_old
    for j in range(NCH):
        M = jnp.maximum(M, chunk(j))

    ls, onehots = [], []
    for t in range(TOPK):
        v = jnp.max(M, axis=0, keepdims=True)
        l = jnp.min(jnp.where(M == v, sub, 128), axis=0, keepdims=True)
        oh = sub == l
        ls.append(l)
        onehots.append(oh)
        M = jnp.where(oh, NEG_INF, M)

    amasks = [jnp.where(oh, 0.0, NEG_INF) for oh in onehots]
    cvals = [[None] * (NCH + 1) for _ in range(TOPK)]
    for j in range(NCH + 1):
        c = chunk(j)
        for t in range(TOPK):
            cvals[t][j] = jnp.max(c + amasks[t], axis=0, keepdims=True)
    rows_v, rows_i = [], []
    for t in range(TOPK):
        ri_c = jnp.sum(jnp.where(onehots[t], ri_old, 0),
                       axis=0, keepdims=True)
        for j in range(NCH + 1):
            rows_v.append(cvals[t][j])
            rows_i.append(pid * KB + j * 128 + ls[t] if j < RUNCH else ri_c)
    NC2 = TOPK * (NCH + 1)
    CV = jnp.concatenate(rows_v, axis=0)
    CI = jnp.concatenate(rows_i, axis=0)
    sub2 = lax.broadcasted_iota(jnp.int32, (NC2, B), 0)

    nv = jnp.full((128, B), NEG_INF, jnp.float32)
    ni = jnp.zeros((128, B), jnp.int32)
    for t in range(TOPK):
        v = jnp.max(CV, axis=0, keepdims=True)
        p = jnp.min(jnp.where(CV == v, sub2, NC2), axis=0, keepdims=True)
        oh2 = sub2 == p
        gidx = jnp.sum(jnp.where(oh2, CI, 0), axis=0, keepdims=True)
        nv = jnp.where(sub == t, v, nv)
        ni = jnp.where(sub == t, gidx, ni)
        CV = jnp.where(oh2, NEG_INF, CV)

    run_vals[...] = nv
    run_idx[...] = ni

    @pl.when(pid == NBLK - 1)
    def _fini():
        finite = (nv > NEG_INF) & (sub < TOPK)
        valid = jnp.where(finite, 1.0, 0.0)
        denom = jnp.maximum(jnp.sum(valid, axis=0, keepdims=True), 1.0)
        w = valid / denom
        idx_out[...] = ni[:8, :]
        w_out[...] = w[:8, :]


def _masked_topk(content, keys, spk, tgt):
    spk2 = spk.reshape(K, 1)
    tgt2 = tgt.reshape(1, B)
    return pl.pallas_call(
        _topk_body,
        grid=(NBLK,),
        in_specs=[
            pl.BlockSpec((B, D), lambda i: (0, 0)),
            pl.BlockSpec((KB, D), lambda i: (i, 0)),
            pl.BlockSpec((KB, 1), lambda i: (i, 0)),
            pl.BlockSpec((1, B), lambda i: (0, 0)),
        ],
        out_specs=[
            pl.BlockSpec((8, B), lambda i: (0, 0)),
            pl.BlockSpec((8, B), lambda i: (0, 0)),
        ],
        out_shape=[
            jax.ShapeDtypeStruct((8, B), jnp.int32),
            jax.ShapeDtypeStruct((8, B), jnp.float32),
        ],
        scratch_shapes=[
            pltpu.VMEM((128, B), jnp.float32),
            pltpu.VMEM((128, B), jnp.int32),
            pltpu.VMEM((B, D), jnp.float32),
        ],
        compiler_params=pltpu.CompilerParams(
            dimension_semantics=("arbitrary",)),
    )(content, keys, spk2, tgt2)


def _sc_gather(table, idx_flat):
    info = plsc.get_sparse_core_info()
    nc, ns = info.num_cores, info.num_subcores
    nw = nc * ns
    n = idx_flat.shape[0]
    b_per_w = n // nw
    mesh = plsc.VectorSubcoreMesh(core_axis_name="c", subcore_axis_name="s")

    @functools.partial(
        pl.kernel,
        mesh=mesh,
        out_type=jax.ShapeDtypeStruct((n, D), jnp.float32),
        scratch_types=[
            pltpu.VMEM((b_per_w,), jnp.int32),
            pltpu.VMEM((b_per_w, D), jnp.float32),
            pltpu.SemaphoreType.DMA,
        ],
    )
    def gather_kernel(table_hbm, idx_hbm, out_hbm, idx_v, rows_v, sem):
        wid = lax.axis_index("s") * nc + lax.axis_index("c")
        base = wid * b_per_w
        pltpu.sync_copy(idx_hbm.at[pl.ds(base, b_per_w)], idx_v)
        pltpu.async_copy(table_hbm.at[idx_v], rows_v, sem).wait()
        pltpu.sync_copy(rows_v, out_hbm.at[pl.ds(base, b_per_w)])

    return gather_kernel(table, idx_flat)


def _mlp_body(content_ref, g_ref, w_ref, W1_ref, b1_ref, W2_ref, b2_ref,
              out_ref):
    content = content_ref[...]
    w8 = w_ref[...]
    rmean = jnp.zeros((B, D), jnp.float32)
    sumw = jnp.zeros((B, 1), jnp.float32)
    for t in range(TOPK):
        wt = w8[t].reshape(B, 1)
        rmean = rmean + g_ref[t] * wt
        sumw = sumw + wt
    has_any = sumw > 0.5

    W1 = W1_ref[...]
    h = lax.dot_general(content, W1[:, :D], (((1,), (1,)), ((), ())),
                        preferred_element_type=jnp.float32)
    h = h + lax.dot_general(rmean, W1[:, D:], (((1,), (1,)), ((), ())),
                            preferred_element_type=jnp.float32)
    h = jnp.maximum(h + b1_ref[...], 0.0)
    out = lax.dot_general(h, W2_ref[...], (((1,), (1,)), ((), ())),
                          preferred_element_type=jnp.float32)
    out = out + b2_ref[...]
    out_ref[...] = jnp.where(has_any, out, content)


def _mlp(content, gathered, w8, W1, b1, W2, b2):
    return pl.pallas_call(
        _mlp_body,
        out_shape=jax.ShapeDtypeStruct((B, D), jnp.float32),
    )(content, gathered, w8, W1, b1.reshape(1, D), W2, b2.reshape(1, D))


@jax.jit
def kernel(content_features, training_features, W1, b1, W2, b2,
           target_speaker_id, speaker_ids):
    top_idx8, top_w8 = _masked_topk(content_features, training_features,
                                    speaker_ids.astype(jnp.int32),
                                    target_speaker_id.astype(jnp.int32))
    idx_flat = top_idx8[:TOPK].reshape(B * TOPK)
    gathered = _sc_gather(training_features, idx_flat)
    g3 = gathered.reshape(TOPK, B, D)
    return _mlp(content_features, g3, top_w8, W1, b1, W2, b2)


# speedup vs baseline: 1.0090x; 1.0041x over previous
import functools

import jax
import jax.numpy as jnp
from jax import lax
from jax.experimental import pallas as pl
from jax.experimental.pallas import tpu as pltpu
from jax.experimental.pallas import tpu_sc as plsc

B, D, K, TOPK = 256, 768, 100000, 5
KB = 4000
NBLK = K // KB
EPS = 1e-8
NEG_INF = float("-inf")

NFULL = KB // 128
TAILW = KB - NFULL * 128
NCH = NFULL + (1 if TAILW else 0)
RUNCH = NCH


def _topk_body(content_ref, keys_ref, spk_ref, tgt_ref, idx_out, w_out,
               run_vals, run_idx, qn_s):
    pid = pl.program_id(0)

    @pl.when(pid == 0)
    def _init():
        run_vals[...] = jnp.full((128, B), NEG_INF, jnp.float32)
        run_idx[...] = jnp.zeros((128, B), jnp.int32)
        content = content_ref[...]
        qn_s[...] = content / (jnp.sqrt(jnp.sum(content * content, axis=1,
                                                keepdims=True)) + EPS)

    qn = qn_s[...]
    keys = keys_ref[...]
    ssq = jnp.sum(keys * keys, axis=1, keepdims=True)
    kn = keys / (jnp.sqrt(ssq) + EPS)

    sims = lax.dot_general(kn, qn, (((1,), (1,)), ((), ())),
                           preferred_element_type=jnp.float32)

    spk = spk_ref[...]
    tgt = tgt_ref[...]
    masked = jnp.where(spk == tgt, sims, NEG_INF)

    run_old = run_vals[...]
    ri_old = run_idx[...]
    sub = lax.broadcasted_iota(jnp.int32, (128, B), 0)

    def chunk(j):
        if j == RUNCH:
            return run_old
        c = masked[j * 128:min((j + 1) * 128, KB), :]
        if c.shape[0] < 128:
            c = jnp.concatenate(
                [c, jnp.full((128 - c.shape[0], B), NEG_INF, jnp.float32)],
                axis=0)
        return c

    M = run_old
    G = jnp.full((128, B), RUNCH, jnp.int32)
    for j in range(NCH):
        c = chunk(j)
        upd = c > M
        M = jnp.where(upd, c, M)
        G = jnp.where(upd, j, G)

    nv = jnp.full((128, B), NEG_INF, jnp.float32)
    ni = jnp.zeros((128, B), jnp.int32)
    for t in range(TOPK):
        v = jnp.max(M, axis=0, keepdims=True)
        eq = M == v
        l = jnp.min(jnp.where(eq, sub, 128), axis=0, keepdims=True)
        onehot = sub == l
        g = jnp.sum(jnp.where(onehot, G, 0), axis=0, keepdims=True)
        ri_sel = jnp.sum(jnp.where(onehot, ri_old, 0), axis=0, keepdims=True)
        gidx = jnp.where(g < RUNCH, pid * KB + g * 128 + l, ri_sel)
        nv = jnp.where(sub == t, v, nv)
        ni = jnp.where(sub == t, gidx, ni)
        best = jnp.full((1, B), NEG_INF, jnp.float32)
        bestj = jnp.full((1, B), RUNCH, jnp.int32)
        for j in range(NCH + 1):
            colv = jnp.max(jnp.where(onehot, chunk(j), NEG_INF),
                           axis=0, keepdims=True)
            colv = jnp.where(colv < v, colv, NEG_INF)
            upd = colv > best
            best = jnp.where(upd, colv, best)
            bestj = jnp.where(upd, j, bestj)
        M = jnp.where(onehot, best, M)
        G = jnp.where(onehot, bestj, G)

    run_vals[...] = nv
    run_idx[...] = ni

    @pl.when(pid == NBLK - 1)
    def _fini():
        finite = (nv > NEG_INF) & (sub < TOPK)
        valid = jnp.where(finite, 1.0, 0.0)
        denom = jnp.maximum(jnp.sum(valid, axis=0, keepdims=True), 1.0)
        w = valid / denom
        idx_out[...] = ni[:8, :]
        w_out[...] = w[:8, :]


def _masked_topk(content, keys, spk, tgt):
    spk2 = spk.reshape(K, 1)
    tgt2 = tgt.reshape(1, B)
    return pl.pallas_call(
        _topk_body,
        grid=(NBLK,),
        in_specs=[
            pl.BlockSpec((B, D), lambda i: (0, 0)),
            pl.BlockSpec((KB, D), lambda i: (i, 0)),
            pl.BlockSpec((KB, 1), lambda i: (i, 0)),
            pl.BlockSpec((1, B), lambda i: (0, 0)),
        ],
        out_specs=[
            pl.BlockSpec((8, B), lambda i: (0, 0)),
            pl.BlockSpec((8, B), lambda i: (0, 0)),
        ],
        out_shape=[
            jax.ShapeDtypeStruct((8, B), jnp.int32),
            jax.ShapeDtypeStruct((8, B), jnp.float32),
        ],
        scratch_shapes=[
            pltpu.VMEM((128, B), jnp.float32),
            pltpu.VMEM((128, B), jnp.int32),
            pltpu.VMEM((B, D), jnp.float32),
        ],
        compiler_params=pltpu.CompilerParams(
            dimension_semantics=("arbitrary",)),
    )(content, keys, spk2, tgt2)


def _sc_gather(table, idx_flat):
    info = plsc.get_sparse_core_info()
    nc, ns = info.num_cores, info.num_subcores
    nw = nc * ns
    n = idx_flat.shape[0]
    b_per_w = n // nw
    mesh = plsc.VectorSubcoreMesh(core_axis_name="c", subcore_axis_name="s")

    @functools.partial(
        pl.kernel,
        mesh=mesh,
        out_type=jax.ShapeDtypeStruct((n, D), jnp.float32),
        scratch_types=[
            pltpu.VMEM((b_per_w,), jnp.int32),
            pltpu.VMEM((b_per_w, D), jnp.float32),
            pltpu.SemaphoreType.DMA,
        ],
    )
    def gather_kernel(table_hbm, idx_hbm, out_hbm, idx_v, rows_v, sem):
        wid = lax.axis_index("s") * nc + lax.axis_index("c")
        base = wid * b_per_w
        pltpu.sync_copy(idx_hbm.at[pl.ds(base, b_per_w)], idx_v)
        pltpu.async_copy(table_hbm.at[idx_v], rows_v, sem).wait()
        pltpu.sync_copy(rows_v, out_hbm.at[pl.ds(base, b_per_w)])

    return gather_kernel(table, idx_flat)


def _mlp_body(content_ref, g_ref, w_ref, W1_ref, b1_ref, W2_ref, b2_ref,
              out_ref):
    content = content_ref[...]
    w8 = w_ref[...]
    rmean = jnp.zeros((B, D), jnp.float32)
    sumw = jnp.zeros((B, 1), jnp.float32)
    for t in range(TOPK):
        wt = w8[t].reshape(B, 1)
        rmean = rmean + g_ref[t] * wt
        sumw = sumw + wt
    has_any = sumw > 0.5

    W1 = W1_ref[...]
    h = lax.dot_general(content, W1[:, :D], (((1,), (1,)), ((), ())),
                        preferred_element_type=jnp.float32)
    h = h + lax.dot_general(rmean, W1[:, D:], (((1,), (1,)), ((), ())),
                            preferred_element_type=jnp.float32)
    h = jnp.maximum(h + b1_ref[...], 0.0)
    out = lax.dot_general(h, W2_ref[...], (((1,), (1,)), ((), ())),
                          preferred_element_type=jnp.float32)
    out = out + b2_ref[...]
    out_ref[...] = jnp.where(has_any, out, content)


def _mlp(content, gathered, w8, W1, b1, W2, b2):
    return pl.pallas_call(
        _mlp_body,
        out_shape=jax.ShapeDtypeStruct((B, D), jnp.float32),
    )(content, gathered, w8, W1, b1.reshape(1, D), W2, b2.reshape(1, D))


@jax.jit
def kernel(content_features, training_features, W1, b1, W2, b2,
           target_speaker_id, speaker_ids):
    top_idx8, top_w8 = _masked_topk(content_features, training_features,
                                    speaker_ids.astype(jnp.int32),
                                    target_speaker_id.astype(jnp.int32))
    idx_flat = top_idx8[:TOPK].reshape(B * TOPK)
    gathered = _sc_gather(training_features, idx_flat)
    g3 = gathered.reshape(TOPK, B, D)
    return _mlp(content_features, g3, top_w8, W1, b1, W2, b2)
